# baseline (speedup 1.0000x reference)
import functools

import jax
import jax.numpy as jnp
from jax import lax
from jax.experimental import pallas as pl
from jax.experimental.pallas import tpu as pltpu
from jax.experimental.pallas import tpu_sc as plsc

N_NODES = 10000
N_EDGES = 320000
D_IN = 128
D_HID = 128
D_OUT = 64

NC = 2
NS = 16
NW = NC * NS
B = 128
NB = 3
T1 = 2 * NB * -(-N_EDGES // (NS * B * 2 * NB))
T2 = T1 // 2
E_PAD = NS * T1 * B
N_PAD = 10240
RPS = N_PAD // NS
ZCH = RPS // B
HA = 80


def _sc_agg_body(H, T, col_split, *refs):
    (z_hbm, src_hbm, dst_hbm, zrows_hbm,
     acc_out, srcb, dstb, rows, acc_sh, gsem0, gsem1, gsem2) = refs
    c = lax.axis_index("c")
    s = lax.axis_index("s")

    pltpu.sync_copy(zrows_hbm, rows.at[0])

    def zbody(i, _):
        pltpu.sync_copy(rows.at[0], acc_sh.at[pl.ds(s * RPS + i * B, B)])
        return _

    lax.fori_loop(0, ZCH, zbody, None)

    chunk = s if col_split else s * NC + c
    pltpu.sync_copy(src_hbm.at[chunk], srcb)
    pltpu.sync_copy(dst_hbm.at[chunk], dstb)
    plsc.subcore_barrier()

    gather_src = z_hbm.at[c] if col_split else z_hbm

    gsems = (gsem0, gsem1, gsem2)

    def ebody(i, _):
        j = NB * i
        ds = [
            pltpu.async_copy(
                gather_src.at[srcb.at[j + b]], rows.at[b], gsems[b])
            for b in range(NB)
        ]
        for b in range(NB):
            ds[b].wait()
            pltpu.sync_copy(rows.at[b], acc_sh.at[dstb.at[j + b]], add=True)
        return _

    lax.fori_loop(0, T // NB, ebody, None)
    plsc.subcore_barrier()

    def obody(i, _):
        r0 = s * RPS + i * B
        pltpu.sync_copy(acc_sh.at[pl.ds(r0, B)], rows.at[0])
        pltpu.sync_copy(rows.at[0], acc_out.at[c].at[pl.ds(r0, B)])
        return _

    lax.fori_loop(0, ZCH, obody, None)


def _make_sc_agg(H, T, col_split):
    mesh = plsc.VectorSubcoreMesh(core_axis_name="c", subcore_axis_name="s")
    return pl.kernel(
        functools.partial(_sc_agg_body, H, T, col_split),
        out_type=(jax.ShapeDtypeStruct((NC, N_PAD, H), jnp.float32),),
        mesh=mesh,
        scratch_types=(
            pltpu.VMEM((T, B), jnp.int32),
            pltpu.VMEM((T, B), jnp.int32),
            pltpu.VMEM((NB, B, H), jnp.float32),
            pltpu.VMEM_SHARED((N_PAD, H), jnp.float32),
            pltpu.SemaphoreType.DMA,
            pltpu.SemaphoreType.DMA,
            pltpu.SemaphoreType.DMA,
        ),
        compiler_params=pltpu.CompilerParams(use_tc_tiling_on_sc=False),
    )


def _tc_mid_body(x_ref, acc_ref, ws1, wn1, b1, ws2, wn2, b2,
                 z2_ref, s2_ref):
    x = x_ref[...]
    agg = jnp.concatenate([acc_ref[0, :, :64], acc_ref[1, :, :64]], axis=-1)
    deg = jnp.maximum(acc_ref[0, :, 64:65], 1.0)
    h_n = agg / deg
    dot = functools.partial(jnp.dot, precision=lax.Precision.HIGHEST,
                            preferred_element_type=jnp.float32)
    h1 = dot(x, ws1[...]) + dot(h_n, wn1[...]) + b1[...]
    h1 = jnp.maximum(h1, 0.0)
    z2_ref[...] = dot(h1, wn2[...])
    s2_ref[...] = dot(h1, ws2[...]) + b2[...]


def _tc_out_body(acc2_ref, acc1_ref, s2_ref, out_ref):
    agg = acc2_ref[0] + acc2_ref[1]
    deg = jnp.maximum(acc1_ref[0, :, 64:65], 1.0)
    out_ref[...] = s2_ref[...] + agg / deg


_R = 1000


def kernel(node_features, edge_index, W_self1, W_neigh1, b1,
           W_self2, W_neigh2, b2):
    src = edge_index[0].astype(jnp.int32)
    dst = edge_index[1].astype(jnp.int32)
    pad = E_PAD - N_EDGES
    src_p = jnp.concatenate([src, jnp.zeros((pad,), jnp.int32)])
    dst_p = jnp.concatenate([dst, jnp.full((pad,), N_NODES, jnp.int32)])
    src_a, dst_a = src_p.reshape(NS, T1, B), dst_p.reshape(NS, T1, B)
    src_b, dst_b = src_p.reshape(NW, T2, B), dst_p.reshape(NW, T2, B)

    one = jnp.ones((N_NODES, 1), jnp.float32)
    zpad = jnp.zeros((N_NODES, HA - 65), jnp.float32)
    x_aug = jnp.stack([
        jnp.concatenate([node_features[:, :64], one, zpad], axis=1),
        jnp.concatenate([node_features[:, 64:], one, zpad], axis=1),
    ])

    (acc1,) = _make_sc_agg(HA, T1, True)(
        x_aug, src_a, dst_a, jnp.zeros((B, HA), jnp.float32))

    grid = N_NODES // _R
    full = lambda i: (0, 0)
    z2, s2 = pl.pallas_call(
        _tc_mid_body,
        grid=(grid,),
        in_specs=[
            pl.BlockSpec((_R, D_IN), lambda i: (i, 0)),
            pl.BlockSpec((NC, _R, HA), lambda i: (0, i, 0)),
            pl.BlockSpec((D_IN, D_HID), full),
            pl.BlockSpec((D_IN, D_HID), full),
            pl.BlockSpec((1, D_HID), full),
            pl.BlockSpec((D_HID, D_OUT), full),
            pl.BlockSpec((D_HID, D_OUT), full),
            pl.BlockSpec((1, D_OUT), full),
        ],
        out_specs=[
            pl.BlockSpec((_R, D_OUT), lambda i: (i, 0)),
            pl.BlockSpec((_R, D_OUT), lambda i: (i, 0)),
        ],
        out_shape=[
            jax.ShapeDtypeStruct((N_NODES, D_OUT), jnp.float32),
            jax.ShapeDtypeStruct((N_NODES, D_OUT), jnp.float32),
        ],
    )(node_features, acc1, W_self1, W_neigh1, b1.reshape(1, D_HID),
      W_self2, W_neigh2, b2.reshape(1, D_OUT))

    (acc2,) = _make_sc_agg(D_OUT, T2, False)(
        z2, src_b, dst_b, jnp.zeros((B, D_OUT), jnp.float32))

    out = pl.pallas_call(
        _tc_out_body,
        grid=(grid,),
        in_specs=[
            pl.BlockSpec((NC, _R, D_OUT), lambda i: (0, i, 0)),
            pl.BlockSpec((NC, _R, HA), lambda i: (0, i, 0)),
            pl.BlockSpec((_R, D_OUT), lambda i: (i, 0)),
        ],
        out_specs=pl.BlockSpec((_R, D_OUT), lambda i: (i, 0)),
        out_shape=jax.ShapeDtypeStruct((N_NODES, D_OUT), jnp.float32),
    )(acc2, acc1, s2)
    return out


# speedup vs baseline: 1.2242x; 1.2242x over previous
import functools

import jax
import jax.numpy as jnp
from jax import lax
from jax.experimental import pallas as pl
from jax.experimental.pallas import tpu as pltpu
from jax.experimental.pallas import tpu_sc as plsc

N_NODES = 10000
N_EDGES = 320000
D_IN = 128
D_HID = 128
D_OUT = 64

NC = 2
NS = 16
NW = NC * NS
B = 128
NB = 2
T1 = 2 * NB * -(-N_EDGES // (NS * B * 2 * NB))
T2 = T1 // 2
E_PAD = NS * T1 * B
N_PAD = 10240
RPS = N_PAD // NS
ZCH = RPS // B
HA = 80


def _sc_agg_body(H, T, col_split, *refs):
    (z_hbm, src_hbm, dst_hbm, zrows_hbm,
     acc_out, srcb, dstb, rows, acc_sh, gsem0, gsem1) = refs
    c = lax.axis_index("c")
    s = lax.axis_index("s")

    pltpu.sync_copy(zrows_hbm, rows.at[0])

    def zbody(i, _):
        pltpu.sync_copy(rows.at[0], acc_sh.at[pl.ds(s * RPS + i * B, B)])
        return _

    lax.fori_loop(0, ZCH, zbody, None)

    chunk = s if col_split else s * NC + c
    pltpu.sync_copy(src_hbm.at[chunk], srcb)
    pltpu.sync_copy(dst_hbm.at[chunk], dstb)
    plsc.subcore_barrier()

    gather_src = z_hbm.at[c] if col_split else z_hbm

    gsems = (gsem0, gsem1)

    def ebody(i, _):
        j = NB * i
        ds = [
            pltpu.async_copy(
                gather_src.at[srcb.at[j + b]], rows.at[b], gsems[b])
            for b in range(NB)
        ]
        for b in range(NB):
            ds[b].wait()
            pltpu.sync_copy(rows.at[b], acc_sh.at[dstb.at[j + b]], add=True)
        return _

    lax.fori_loop(0, T // NB, ebody, None)
    plsc.subcore_barrier()

    def obody(i, _):
        r0 = s * RPS + i * B
        pltpu.sync_copy(acc_sh.at[pl.ds(r0, B)], rows.at[0])
        pltpu.sync_copy(rows.at[0], acc_out.at[c].at[pl.ds(r0, B)])
        return _

    lax.fori_loop(0, ZCH, obody, None)


def _make_sc_agg(H, T, col_split):
    mesh = plsc.VectorSubcoreMesh(core_axis_name="c", subcore_axis_name="s")
    return pl.kernel(
        functools.partial(_sc_agg_body, H, T, col_split),
        out_type=(jax.ShapeDtypeStruct((NC, N_PAD, H), jnp.float32),),
        mesh=mesh,
        scratch_types=(
            pltpu.VMEM((T, B), jnp.int32),
            pltpu.VMEM((T, B), jnp.int32),
            pltpu.VMEM((NB, B, H), jnp.float32),
            pltpu.VMEM_SHARED((N_PAD, H), jnp.float32),
            pltpu.SemaphoreType.DMA,
            pltpu.SemaphoreType.DMA,
        ),
        compiler_params=pltpu.CompilerParams(use_tc_tiling_on_sc=False),
    )


def _tc_mid_body(x_ref, acc_ref, ws1, wn1, b1, ws2, wn2, b2,
                 z2_ref, s2_ref):
    x = x_ref[...]
    agg = jnp.concatenate([acc_ref[0, :, :64], acc_ref[1, :, :64]], axis=-1)
    deg = jnp.maximum(acc_ref[0, :, 64:65], 1.0)
    h_n = agg / deg
    dot = functools.partial(jnp.dot, precision=lax.Precision.HIGHEST,
                            preferred_element_type=jnp.float32)
    h1 = dot(x, ws1[...]) + dot(h_n, wn1[...]) + b1[...]
    h1 = jnp.maximum(h1, 0.0)
    z2_ref[...] = dot(h1, wn2[...])
    s2_ref[...] = dot(h1, ws2[...]) + b2[...]


def _tc_out_body(acc2_ref, acc1_ref, s2_ref, out_ref):
    agg = acc2_ref[0] + acc2_ref[1]
    deg = jnp.maximum(acc1_ref[0, :, 64:65], 1.0)
    out_ref[...] = s2_ref[...] + agg / deg


_R = 1000


def kernel(node_features, edge_index, W_self1, W_neigh1, b1,
           W_self2, W_neigh2, b2):
    src = edge_index[0].astype(jnp.int32)
    dst = edge_index[1].astype(jnp.int32)
    pad = E_PAD - N_EDGES
    src_p = jnp.concatenate([src, jnp.zeros((pad,), jnp.int32)])
    dst_p = jnp.concatenate([dst, jnp.full((pad,), N_NODES, jnp.int32)])
    src_a, dst_a = src_p.reshape(NS, T1, B), dst_p.reshape(NS, T1, B)
    src_b, dst_b = src_p.reshape(NW, T2, B), dst_p.reshape(NW, T2, B)

    one = jnp.ones((N_NODES, 1), jnp.float32)
    zpad = jnp.zeros((N_NODES, HA - 65), jnp.float32)
    x_aug = jnp.stack([
        jnp.concatenate([node_features[:, :64], one, zpad], axis=1),
        jnp.concatenate([node_features[:, 64:], one, zpad], axis=1),
    ])

    (acc1,) = _make_sc_agg(HA, T1, True)(
        x_aug, src_a, dst_a, jnp.zeros((B, HA), jnp.float32))

    grid = N_NODES // _R
    full = lambda i: (0, 0)
    z2, s2 = pl.pallas_call(
        _tc_mid_body,
        grid=(grid,),
        in_specs=[
            pl.BlockSpec((_R, D_IN), lambda i: (i, 0)),
            pl.BlockSpec((NC, _R, HA), lambda i: (0, i, 0)),
            pl.BlockSpec((D_IN, D_HID), full),
            pl.BlockSpec((D_IN, D_HID), full),
            pl.BlockSpec((1, D_HID), full),
            pl.BlockSpec((D_HID, D_OUT), full),
            pl.BlockSpec((D_HID, D_OUT), full),
            pl.BlockSpec((1, D_OUT), full),
        ],
        out_specs=[
            pl.BlockSpec((_R, D_OUT), lambda i: (i, 0)),
            pl.BlockSpec((_R, D_OUT), lambda i: (i, 0)),
        ],
        out_shape=[
            jax.ShapeDtypeStruct((N_NODES, D_OUT), jnp.float32),
            jax.ShapeDtypeStruct((N_NODES, D_OUT), jnp.float32),
        ],
    )(node_features, acc1, W_self1, W_neigh1, b1.reshape(1, D_HID),
      W_self2, W_neigh2, b2.reshape(1, D_OUT))

    (acc2,) = _make_sc_agg(D_OUT, T2, False)(
        z2, src_b, dst_b, jnp.zeros((B, D_OUT), jnp.float32))

    out = pl.pallas_call(
        _tc_out_body,
        grid=(grid,),
        in_specs=[
            pl.BlockSpec((NC, _R, D_OUT), lambda i: (0, i, 0)),
            pl.BlockSpec((NC, _R, HA), lambda i: (0, i, 0)),
            pl.BlockSpec((_R, D_OUT), lambda i: (i, 0)),
        ],
        out_specs=pl.BlockSpec((_R, D_OUT), lambda i: (i, 0)),
        out_shape=jax.ShapeDtypeStruct((N_NODES, D_OUT), jnp.float32),
    )(acc2, acc1, s2)
    return out


# speedup vs baseline: 1.4623x; 1.1945x over previous
import functools

import jax
import jax.numpy as jnp
from jax import lax
from jax.experimental import pallas as pl
from jax.experimental.pallas import tpu as pltpu
from jax.experimental.pallas import tpu_sc as plsc

N_NODES = 10000
N_EDGES = 320000
D_IN = 128
D_HID = 128
D_OUT = 64

NC = 2
NS = 16
NW = NC * NS
B = 128
NB = 1
T1 = 2 * NB * -(-N_EDGES // (NS * B * 2 * NB))
T2 = T1 // 2
E_PAD = NS * T1 * B
N_PAD = 10240
RPS = N_PAD // NS
ZCH = RPS // B
HA = 80


def _sc_agg_body(H, T, col_split, *refs):
    (z_hbm, src_hbm, dst_hbm, zrows_hbm,
     acc_out, srcb, dstb, rows, acc_sh, gsem0) = refs
    c = lax.axis_index("c")
    s = lax.axis_index("s")

    pltpu.sync_copy(zrows_hbm, rows.at[0])

    def zbody(i, _):
        pltpu.sync_copy(rows.at[0], acc_sh.at[pl.ds(s * RPS + i * B, B)])
        return _

    lax.fori_loop(0, ZCH, zbody, None)

    chunk = s if col_split else s * NC + c
    pltpu.sync_copy(src_hbm.at[chunk], srcb)
    pltpu.sync_copy(dst_hbm.at[chunk], dstb)
    plsc.subcore_barrier()

    gather_src = z_hbm.at[c] if col_split else z_hbm

    def ebody(j, _):
        pltpu.async_copy(gather_src.at[srcb.at[j]], rows.at[0], gsem0).wait()
        pltpu.sync_copy(rows.at[0], acc_sh.at[dstb.at[j]], add=True)
        return _

    lax.fori_loop(0, T, ebody, None)
    plsc.subcore_barrier()

    def obody(i, _):
        r0 = s * RPS + i * B
        pltpu.sync_copy(acc_sh.at[pl.ds(r0, B)], rows.at[0])
        pltpu.sync_copy(rows.at[0], acc_out.at[c].at[pl.ds(r0, B)])
        return _

    lax.fori_loop(0, ZCH, obody, None)


def _make_sc_agg(H, T, col_split):
    mesh = plsc.VectorSubcoreMesh(core_axis_name="c", subcore_axis_name="s")
    return pl.kernel(
        functools.partial(_sc_agg_body, H, T, col_split),
        out_type=(jax.ShapeDtypeStruct((NC, N_PAD, H), jnp.float32),),
        mesh=mesh,
        scratch_types=(
            pltpu.VMEM((T, B), jnp.int32),
            pltpu.VMEM((T, B), jnp.int32),
            pltpu.VMEM((NB, B, H), jnp.float32),
            pltpu.VMEM_SHARED((N_PAD, H), jnp.float32),
            pltpu.SemaphoreType.DMA,
        ),
        compiler_params=pltpu.CompilerParams(use_tc_tiling_on_sc=False),
    )


def _tc_mid_body(x_ref, acc_ref, ws1, wn1, b1, ws2, wn2, b2,
                 z2_ref, s2_ref):
    x = x_ref[...]
    agg = jnp.concatenate([acc_ref[0, :, :64], acc_ref[1, :, :64]], axis=-1)
    deg = jnp.maximum(acc_ref[0, :, 64:65], 1.0)
    h_n = agg / deg
    dot = functools.partial(jnp.dot, precision=lax.Precision.HIGHEST,
                            preferred_element_type=jnp.float32)
    h1 = dot(x, ws1[...]) + dot(h_n, wn1[...]) + b1[...]
    h1 = jnp.maximum(h1, 0.0)
    z2_ref[...] = dot(h1, wn2[...])
    s2_ref[...] = dot(h1, ws2[...]) + b2[...]


def _tc_out_body(acc2_ref, acc1_ref, s2_ref, out_ref):
    agg = acc2_ref[0] + acc2_ref[1]
    deg = jnp.maximum(acc1_ref[0, :, 64:65], 1.0)
    out_ref[...] = s2_ref[...] + agg / deg


_R = 1000


def kernel(node_features, edge_index, W_self1, W_neigh1, b1,
           W_self2, W_neigh2, b2):
    src = edge_index[0].astype(jnp.int32)
    dst = edge_index[1].astype(jnp.int32)
    pad = E_PAD - N_EDGES
    src_p = jnp.concatenate([src, jnp.zeros((pad,), jnp.int32)])
    dst_p = jnp.concatenate([dst, jnp.full((pad,), N_NODES, jnp.int32)])
    src_a, dst_a = src_p.reshape(NS, T1, B), dst_p.reshape(NS, T1, B)
    src_b, dst_b = src_p.reshape(NW, T2, B), dst_p.reshape(NW, T2, B)

    one = jnp.ones((N_NODES, 1), jnp.float32)
    zpad = jnp.zeros((N_NODES, HA - 65), jnp.float32)
    x_aug = jnp.stack([
        jnp.concatenate([node_features[:, :64], one, zpad], axis=1),
        jnp.concatenate([node_features[:, 64:], one, zpad], axis=1),
    ])

    (acc1,) = _make_sc_agg(HA, T1, True)(
        x_aug, src_a, dst_a, jnp.zeros((B, HA), jnp.float32))

    grid = N_NODES // _R
    full = lambda i: (0, 0)
    z2, s2 = pl.pallas_call(
        _tc_mid_body,
        grid=(grid,),
        in_specs=[
            pl.BlockSpec((_R, D_IN), lambda i: (i, 0)),
            pl.BlockSpec((NC, _R, HA), lambda i: (0, i, 0)),
            pl.BlockSpec((D_IN, D_HID), full),
            pl.BlockSpec((D_IN, D_HID), full),
            pl.BlockSpec((1, D_HID), full),
            pl.BlockSpec((D_HID, D_OUT), full),
            pl.BlockSpec((D_HID, D_OUT), full),
            pl.BlockSpec((1, D_OUT), full),
        ],
        out_specs=[
            pl.BlockSpec((_R, D_OUT), lambda i: (i, 0)),
            pl.BlockSpec((_R, D_OUT), lambda i: (i, 0)),
        ],
        out_shape=[
            jax.ShapeDtypeStruct((N_NODES, D_OUT), jnp.float32),
            jax.ShapeDtypeStruct((N_NODES, D_OUT), jnp.float32),
        ],
    )(node_features, acc1, W_self1, W_neigh1, b1.reshape(1, D_HID),
      W_self2, W_neigh2, b2.reshape(1, D_OUT))

    (acc2,) = _make_sc_agg(D_OUT, T2, False)(
        z2, src_b, dst_b, jnp.zeros((B, D_OUT), jnp.float32))

    out = pl.pallas_call(
        _tc_out_body,
        grid=(grid,),
        in_specs=[
            pl.BlockSpec((NC, _R, D_OUT), lambda i: (0, i, 0)),
            pl.BlockSpec((NC, _R, HA), lambda i: (0, i, 0)),
            pl.BlockSpec((_R, D_OUT), lambda i: (i, 0)),
        ],
        out_specs=pl.BlockSpec((_R, D_OUT), lambda i: (i, 0)),
        out_shape=jax.ShapeDtypeStruct((N_NODES, D_OUT), jnp.float32),
    )(acc2, acc1, s2)
    return out


# speedup vs baseline: 1.6525x; 1.1301x over previous
import functools

import jax
import jax.numpy as jnp
from jax import lax
from jax.experimental import pallas as pl
from jax.experimental.pallas import tpu as pltpu
from jax.experimental.pallas import tpu_sc as plsc

N_NODES = 10000
N_EDGES = 320000
D_IN = 128
D_HID = 128
D_OUT = 64

NC = 2
NS = 16
NW = NC * NS
B = 128
NB = 1
T1 = 2 * NB * -(-N_EDGES // (NS * B * 2 * NB))
T2 = T1 // 2
E_PAD = NS * T1 * B
N_PAD = 10240
RPS = N_PAD // NS
ZCH = RPS // B
HA = 80


def _sc_agg_body(H, T, col_split, spmem_src, *refs):
    if spmem_src:
        (z_hbm, src_hbm, dst_hbm, zrows_hbm,
         acc_out, srcb, dstb, rows, acc_sh, z_sh, gsem0) = refs
    else:
        (z_hbm, src_hbm, dst_hbm, zrows_hbm,
         acc_out, srcb, dstb, rows, acc_sh, gsem0) = refs
    c = lax.axis_index("c")
    s = lax.axis_index("s")

    pltpu.sync_copy(zrows_hbm, rows.at[0])

    def zbody(i, _):
        pltpu.sync_copy(rows.at[0], acc_sh.at[pl.ds(s * RPS + i * B, B)])
        return _

    lax.fori_loop(0, ZCH, zbody, None)

    if spmem_src:
        def sbody(i, _):
            r0 = s * RPS + i * B
            pltpu.sync_copy(z_hbm.at[pl.ds(r0, B)], rows.at[0])
            pltpu.sync_copy(rows.at[0], z_sh.at[pl.ds(r0, B)])
            return _

        lax.fori_loop(0, ZCH, sbody, None)

    chunk = s if col_split else s * NC + c
    pltpu.sync_copy(src_hbm.at[chunk], srcb)
    pltpu.sync_copy(dst_hbm.at[chunk], dstb)
    plsc.subcore_barrier()

    if spmem_src:
        gather_src = z_sh
    else:
        gather_src = z_hbm.at[c] if col_split else z_hbm

    def ebody(j, _):
        pltpu.async_copy(gather_src.at[srcb.at[j]], rows.at[0], gsem0).wait()
        pltpu.sync_copy(rows.at[0], acc_sh.at[dstb.at[j]], add=True)
        return _

    lax.fori_loop(0, T, ebody, None)
    plsc.subcore_barrier()

    def obody(i, _):
        r0 = s * RPS + i * B
        pltpu.sync_copy(acc_sh.at[pl.ds(r0, B)], rows.at[0])
        pltpu.sync_copy(rows.at[0], acc_out.at[c].at[pl.ds(r0, B)])
        return _

    lax.fori_loop(0, ZCH, obody, None)


def _make_sc_agg(H, T, col_split, spmem_src=False):
    mesh = plsc.VectorSubcoreMesh(core_axis_name="c", subcore_axis_name="s")
    scratch = [
        pltpu.VMEM((T, B), jnp.int32),
        pltpu.VMEM((T, B), jnp.int32),
        pltpu.VMEM((NB, B, H), jnp.float32),
        pltpu.VMEM_SHARED((N_PAD, H), jnp.float32),
    ]
    if spmem_src:
        scratch.append(pltpu.VMEM_SHARED((N_PAD, H), jnp.float32))
    scratch.append(pltpu.SemaphoreType.DMA)
    return pl.kernel(
        functools.partial(_sc_agg_body, H, T, col_split, spmem_src),
        out_type=(jax.ShapeDtypeStruct((NC, N_PAD, H), jnp.float32),),
        mesh=mesh,
        scratch_types=tuple(scratch),
        compiler_params=pltpu.CompilerParams(use_tc_tiling_on_sc=False),
    )


def _tc_mid_body(x_ref, acc_ref, ws1, wn1, b1, ws2, wn2, b2,
                 z2_ref, s2_ref):
    x = x_ref[...]
    agg = jnp.concatenate([acc_ref[0, :, :64], acc_ref[1, :, :64]], axis=-1)
    deg = jnp.maximum(acc_ref[0, :, 64:65], 1.0)
    h_n = agg / deg
    dot = functools.partial(jnp.dot, precision=lax.Precision.HIGHEST,
                            preferred_element_type=jnp.float32)
    h1 = dot(x, ws1[...]) + dot(h_n, wn1[...]) + b1[...]
    h1 = jnp.maximum(h1, 0.0)
    z2_ref[...] = dot(h1, wn2[...])
    s2_ref[...] = dot(h1, ws2[...]) + b2[...]


def _tc_out_body(acc2_ref, acc1_ref, s2_ref, out_ref):
    agg = acc2_ref[0] + acc2_ref[1]
    deg = jnp.maximum(acc1_ref[0, :, 64:65], 1.0)
    out_ref[...] = s2_ref[...] + agg / deg


_R = 1000


def kernel(node_features, edge_index, W_self1, W_neigh1, b1,
           W_self2, W_neigh2, b2):
    src = edge_index[0].astype(jnp.int32)
    dst = edge_index[1].astype(jnp.int32)
    pad = E_PAD - N_EDGES
    src_p = jnp.concatenate([src, jnp.zeros((pad,), jnp.int32)])
    dst_p = jnp.concatenate([dst, jnp.full((pad,), N_NODES, jnp.int32)])
    src_a, dst_a = src_p.reshape(NS, T1, B), dst_p.reshape(NS, T1, B)
    src_b, dst_b = src_p.reshape(NW, T2, B), dst_p.reshape(NW, T2, B)

    one = jnp.ones((N_NODES, 1), jnp.float32)
    zpad = jnp.zeros((N_NODES, HA - 65), jnp.float32)
    x_aug = jnp.stack([
        jnp.concatenate([node_features[:, :64], one, zpad], axis=1),
        jnp.concatenate([node_features[:, 64:], one, zpad], axis=1),
    ])

    (acc1,) = _make_sc_agg(HA, T1, True)(
        x_aug, src_a, dst_a, jnp.zeros((B, HA), jnp.float32))

    grid = N_NODES // _R
    full = lambda i: (0, 0)
    z2, s2 = pl.pallas_call(
        _tc_mid_body,
        grid=(grid,),
        in_specs=[
            pl.BlockSpec((_R, D_IN), lambda i: (i, 0)),
            pl.BlockSpec((NC, _R, HA), lambda i: (0, i, 0)),
            pl.BlockSpec((D_IN, D_HID), full),
            pl.BlockSpec((D_IN, D_HID), full),
            pl.BlockSpec((1, D_HID), full),
            pl.BlockSpec((D_HID, D_OUT), full),
            pl.BlockSpec((D_HID, D_OUT), full),
            pl.BlockSpec((1, D_OUT), full),
        ],
        out_specs=[
            pl.BlockSpec((_R, D_OUT), lambda i: (i, 0)),
            pl.BlockSpec((_R, D_OUT), lambda i: (i, 0)),
        ],
        out_shape=[
            jax.ShapeDtypeStruct((N_NODES, D_OUT), jnp.float32),
            jax.ShapeDtypeStruct((N_NODES, D_OUT), jnp.float32),
        ],
    )(node_features, acc1, W_self1, W_neigh1, b1.reshape(1, D_HID),
      W_self2, W_neigh2, b2.reshape(1, D_OUT))

    z2_pad = jnp.pad(z2, ((0, N_PAD - N_NODES), (0, 0)))
    (acc2,) = _make_sc_agg(D_OUT, T2, False, spmem_src=True)(
        z2_pad, src_b, dst_b, jnp.zeros((B, D_OUT), jnp.float32))

    out = pl.pallas_call(
        _tc_out_body,
        grid=(grid,),
        in_specs=[
            pl.BlockSpec((NC, _R, D_OUT), lambda i: (0, i, 0)),
            pl.BlockSpec((NC, _R, HA), lambda i: (0, i, 0)),
            pl.BlockSpec((_R, D_OUT), lambda i: (i, 0)),
        ],
        out_specs=pl.BlockSpec((_R, D_OUT), lambda i: (i, 0)),
        out_shape=jax.ShapeDtypeStruct((N_NODES, D_OUT), jnp.float32),
    )(acc2, acc1, s2)
    return out


# speedup vs baseline: 1.9049x; 1.1527x over previous
import functools

import jax
import jax.numpy as jnp
from jax import lax
from jax.experimental import pallas as pl
from jax.experimental.pallas import tpu as pltpu
from jax.experimental.pallas import tpu_sc as plsc

N_NODES = 10000
N_EDGES = 320000
D_IN = 128
D_HID = 128
D_OUT = 64

NC = 2
NS = 16
NW = NC * NS
B = 128
NB = 1
GI = 16
T1 = GI * -(-N_EDGES // (NS * B * GI))
T2 = T1 // 2
E_PAD = NS * T1 * B
N_PAD = 10240
RPS = N_PAD // NS
ZCH = RPS // B
HA = 80


def _sc_agg_body(H, T, col_split, spmem_src, *refs):
    if spmem_src:
        (z_hbm, src_hbm, dst_hbm, zrows_hbm,
         acc_out, srcb, dstb, rows, acc_sh, z_sh, gsem0) = refs
    else:
        (z_hbm, src_hbm, dst_hbm, zrows_hbm,
         acc_out, srcb, dstb, rows, acc_sh, gsem0) = refs
    c = lax.axis_index("c")
    s = lax.axis_index("s")

    pltpu.sync_copy(zrows_hbm, rows.at[0])

    def zbody(i, _):
        pltpu.sync_copy(rows.at[0], acc_sh.at[pl.ds(s * RPS + i * B, B)])
        return _

    lax.fori_loop(0, ZCH, zbody, None)

    stage_src = z_hbm.at[c] if col_split else z_hbm
    if spmem_src:
        def sbody(i, _):
            r0 = s * RPS + i * B
            pltpu.sync_copy(stage_src.at[pl.ds(r0, B)], rows.at[0])
            pltpu.sync_copy(rows.at[0], z_sh.at[pl.ds(r0, B)])
            return _

        lax.fori_loop(0, ZCH, sbody, None)
        gather_src = z_sh
    else:
        gather_src = stage_src

    chunk = s if col_split else s * NC + c
    plsc.subcore_barrier()

    def gbody(g, _):
        pltpu.sync_copy(src_hbm.at[chunk].at[pl.ds(g * GI, GI)], srcb)
        pltpu.sync_copy(dst_hbm.at[chunk].at[pl.ds(g * GI, GI)], dstb)

        def ebody(k, _):
            pltpu.async_copy(
                gather_src.at[srcb.at[k]], rows.at[0], gsem0).wait()
            pltpu.sync_copy(rows.at[0], acc_sh.at[dstb.at[k]], add=True)
            return _

        lax.fori_loop(0, GI, ebody, None)
        return _

    lax.fori_loop(0, T // GI, gbody, None)
    plsc.subcore_barrier()

    def obody(i, _):
        r0 = s * RPS + i * B
        pltpu.sync_copy(acc_sh.at[pl.ds(r0, B)], rows.at[0])
        pltpu.sync_copy(rows.at[0], acc_out.at[c].at[pl.ds(r0, B)])
        return _

    lax.fori_loop(0, ZCH, obody, None)


def _make_sc_agg(H, T, col_split, spmem_src=False):
    mesh = plsc.VectorSubcoreMesh(core_axis_name="c", subcore_axis_name="s")
    scratch = [
        pltpu.VMEM((GI, B), jnp.int32),
        pltpu.VMEM((GI, B), jnp.int32),
        pltpu.VMEM((NB, B, H), jnp.float32),
        pltpu.VMEM_SHARED((N_PAD, H), jnp.float32),
    ]
    if spmem_src:
        scratch.append(pltpu.VMEM_SHARED((N_PAD, H), jnp.float32))
    scratch.append(pltpu.SemaphoreType.DMA)
    return pl.kernel(
        functools.partial(_sc_agg_body, H, T, col_split, spmem_src),
        out_type=(jax.ShapeDtypeStruct((NC, N_PAD, H), jnp.float32),),
        mesh=mesh,
        scratch_types=tuple(scratch),
        compiler_params=pltpu.CompilerParams(use_tc_tiling_on_sc=False),
    )


def _tc_mid_body(x_ref, acc_ref, ws1, wn1, b1, ws2, wn2, b2,
                 z2_ref, s2_ref):
    x = x_ref[...]
    agg = jnp.concatenate([acc_ref[0, :, :64], acc_ref[1, :, :64]], axis=-1)
    deg = jnp.maximum(acc_ref[0, :, 64:65], 1.0)
    h_n = agg / deg
    dot = functools.partial(jnp.dot, precision=lax.Precision.HIGHEST,
                            preferred_element_type=jnp.float32)
    h1 = dot(x, ws1[...]) + dot(h_n, wn1[...]) + b1[...]
    h1 = jnp.maximum(h1, 0.0)
    z2_ref[...] = dot(h1, wn2[...])
    s2_ref[...] = dot(h1, ws2[...]) + b2[...]


def _tc_out_body(acc2_ref, acc1_ref, s2_ref, out_ref):
    agg = acc2_ref[0] + acc2_ref[1]
    deg = jnp.maximum(acc1_ref[0, :, 64:65], 1.0)
    out_ref[...] = s2_ref[...] + agg / deg


_R = 1000


def kernel(node_features, edge_index, W_self1, W_neigh1, b1,
           W_self2, W_neigh2, b2):
    src = edge_index[0].astype(jnp.int32)
    dst = edge_index[1].astype(jnp.int32)
    pad = E_PAD - N_EDGES
    src_p = jnp.concatenate([src, jnp.zeros((pad,), jnp.int32)])
    dst_p = jnp.concatenate([dst, jnp.full((pad,), N_NODES, jnp.int32)])
    src_a, dst_a = src_p.reshape(NS, T1, B), dst_p.reshape(NS, T1, B)
    src_b, dst_b = src_p.reshape(NW, T2, B), dst_p.reshape(NW, T2, B)

    one = jnp.ones((N_NODES, 1), jnp.float32)
    zpad = jnp.zeros((N_NODES, HA - 65), jnp.float32)
    x_aug = jnp.stack([
        jnp.concatenate([node_features[:, :64], one, zpad], axis=1),
        jnp.concatenate([node_features[:, 64:], one, zpad], axis=1),
    ])
    x_aug = jnp.pad(x_aug, ((0, 0), (0, N_PAD - N_NODES), (0, 0)))

    (acc1,) = _make_sc_agg(HA, T1, True, spmem_src=True)(
        x_aug, src_a, dst_a, jnp.zeros((B, HA), jnp.float32))

    grid = N_NODES // _R
    full = lambda i: (0, 0)
    z2, s2 = pl.pallas_call(
        _tc_mid_body,
        grid=(grid,),
        in_specs=[
            pl.BlockSpec((_R, D_IN), lambda i: (i, 0)),
            pl.BlockSpec((NC, _R, HA), lambda i: (0, i, 0)),
            pl.BlockSpec((D_IN, D_HID), full),
            pl.BlockSpec((D_IN, D_HID), full),
            pl.BlockSpec((1, D_HID), full),
            pl.BlockSpec((D_HID, D_OUT), full),
            pl.BlockSpec((D_HID, D_OUT), full),
            pl.BlockSpec((1, D_OUT), full),
        ],
        out_specs=[
            pl.BlockSpec((_R, D_OUT), lambda i: (i, 0)),
            pl.BlockSpec((_R, D_OUT), lambda i: (i, 0)),
        ],
        out_shape=[
            jax.ShapeDtypeStruct((N_NODES, D_OUT), jnp.float32),
            jax.ShapeDtypeStruct((N_NODES, D_OUT), jnp.float32),
        ],
    )(node_features, acc1, W_self1, W_neigh1, b1.reshape(1, D_HID),
      W_self2, W_neigh2, b2.reshape(1, D_OUT))

    z2_pad = jnp.pad(z2, ((0, N_PAD - N_NODES), (0, 0)))
    (acc2,) = _make_sc_agg(D_OUT, T2, False, spmem_src=True)(
        z2_pad, src_b, dst_b, jnp.zeros((B, D_OUT), jnp.float32))

    out = pl.pallas_call(
        _tc_out_body,
        grid=(grid,),
        in_specs=[
            pl.BlockSpec((NC, _R, D_OUT), lambda i: (0, i, 0)),
            pl.BlockSpec((NC, _R, HA), lambda i: (0, i, 0)),
            pl.BlockSpec((_R, D_OUT), lambda i: (i, 0)),
        ],
        out_specs=pl.BlockSpec((_R, D_OUT), lambda i: (i, 0)),
        out_shape=jax.ShapeDtypeStruct((N_NODES, D_OUT), jnp.float32),
    )(acc2, acc1, s2)
    return out


# speedup vs baseline: 1.9340x; 1.0153x over previous
import functools

import jax
import jax.numpy as jnp
from jax import lax
from jax.experimental import pallas as pl
from jax.experimental.pallas import tpu as pltpu
from jax.experimental.pallas import tpu_sc as plsc

N_NODES = 10000
N_EDGES = 320000
D_IN = 128
D_HID = 128
D_OUT = 64

NC = 2
NS = 16
NW = NC * NS
B = 128
NB = 1
G1 = 32
T1 = G1 * -(-N_EDGES // (NS * B * G1))
T2 = T1 // 2
G2 = T2
E_PAD = NS * T1 * B
N_PAD = 10240
RPS = N_PAD // NS
ZCH = RPS // B
HA = 80


def _sc_agg_body(H, T, GI, col_split, spmem_src, *refs):
    if spmem_src:
        (z_hbm, src_hbm, dst_hbm, zrows_hbm,
         acc_out, srcb, dstb, rows, acc_sh, z_sh, gsem0) = refs
    else:
        (z_hbm, src_hbm, dst_hbm, zrows_hbm,
         acc_out, srcb, dstb, rows, acc_sh, gsem0) = refs
    c = lax.axis_index("c")
    s = lax.axis_index("s")

    pltpu.sync_copy(zrows_hbm, rows.at[0])

    def zbody(i, _):
        pltpu.sync_copy(rows.at[0], acc_sh.at[pl.ds(s * RPS + i * B, B)])
        return _

    lax.fori_loop(0, ZCH, zbody, None)

    stage_src = z_hbm.at[c] if col_split else z_hbm
    if spmem_src:
        def sbody(i, _):
            r0 = s * RPS + i * B
            pltpu.sync_copy(stage_src.at[pl.ds(r0, B)], rows.at[0])
            pltpu.sync_copy(rows.at[0], z_sh.at[pl.ds(r0, B)])
            return _

        lax.fori_loop(0, ZCH, sbody, None)
        gather_src = z_sh
    else:
        gather_src = stage_src

    chunk = s if col_split else s * NC + c
    plsc.subcore_barrier()

    def gbody(g, _):
        pltpu.sync_copy(src_hbm.at[chunk].at[pl.ds(g * GI, GI)], srcb)
        pltpu.sync_copy(dst_hbm.at[chunk].at[pl.ds(g * GI, GI)], dstb)

        def ebody(k, _):
            pltpu.async_copy(
                gather_src.at[srcb.at[k]], rows.at[0], gsem0).wait()
            pltpu.sync_copy(rows.at[0], acc_sh.at[dstb.at[k]], add=True)
            return _

        lax.fori_loop(0, GI, ebody, None)
        return _

    lax.fori_loop(0, T // GI, gbody, None)
    plsc.subcore_barrier()

    def obody(i, _):
        r0 = s * RPS + i * B
        pltpu.sync_copy(acc_sh.at[pl.ds(r0, B)], rows.at[0])
        pltpu.sync_copy(rows.at[0], acc_out.at[c].at[pl.ds(r0, B)])
        return _

    lax.fori_loop(0, ZCH, obody, None)


def _make_sc_agg(H, T, GI, col_split, spmem_src=False):
    mesh = plsc.VectorSubcoreMesh(core_axis_name="c", subcore_axis_name="s")
    scratch = [
        pltpu.VMEM((GI, B), jnp.int32),
        pltpu.VMEM((GI, B), jnp.int32),
        pltpu.VMEM((NB, B, H), jnp.float32),
        pltpu.VMEM_SHARED((N_PAD, H), jnp.float32),
    ]
    if spmem_src:
        scratch.append(pltpu.VMEM_SHARED((N_PAD, H), jnp.float32))
    scratch.append(pltpu.SemaphoreType.DMA)
    return pl.kernel(
        functools.partial(_sc_agg_body, H, T, GI, col_split, spmem_src),
        out_type=(jax.ShapeDtypeStruct((NC, N_PAD, H), jnp.float32),),
        mesh=mesh,
        scratch_types=tuple(scratch),
        compiler_params=pltpu.CompilerParams(use_tc_tiling_on_sc=False),
    )


def _tc_mid_body(x_ref, acc_ref, ws1, wn1, b1, ws2, wn2, b2,
                 z2_ref, s2_ref):
    x = x_ref[...]
    agg = jnp.concatenate([acc_ref[0, :, :64], acc_ref[1, :, :64]], axis=-1)
    deg = jnp.maximum(acc_ref[0, :, 64:65], 1.0)
    h_n = agg / deg
    dot = functools.partial(jnp.dot, precision=lax.Precision.HIGHEST,
                            preferred_element_type=jnp.float32)
    h1 = dot(x, ws1[...]) + dot(h_n, wn1[...]) + b1[...]
    h1 = jnp.maximum(h1, 0.0)
    z2_ref[...] = dot(h1, wn2[...])
    s2_ref[...] = dot(h1, ws2[...]) + b2[...]


def _tc_out_body(acc2_ref, acc1_ref, s2_ref, out_ref):
    agg = acc2_ref[0] + acc2_ref[1]
    deg = jnp.maximum(acc1_ref[0, :, 64:65], 1.0)
    out_ref[...] = s2_ref[...] + agg / deg


_R = 1000


def kernel(node_features, edge_index, W_self1, W_neigh1, b1,
           W_self2, W_neigh2, b2):
    src = edge_index[0].astype(jnp.int32)
    dst = edge_index[1].astype(jnp.int32)
    pad = E_PAD - N_EDGES
    src_p = jnp.concatenate([src, jnp.zeros((pad,), jnp.int32)])
    dst_p = jnp.concatenate([dst, jnp.full((pad,), N_NODES, jnp.int32)])
    src_a, dst_a = src_p.reshape(NS, T1, B), dst_p.reshape(NS, T1, B)
    src_b, dst_b = src_p.reshape(NW, T2, B), dst_p.reshape(NW, T2, B)

    one = jnp.ones((N_NODES, 1), jnp.float32)
    zpad = jnp.zeros((N_NODES, HA - 65), jnp.float32)
    x_aug = jnp.stack([
        jnp.concatenate([node_features[:, :64], one, zpad], axis=1),
        jnp.concatenate([node_features[:, 64:], one, zpad], axis=1),
    ])
    x_aug = jnp.pad(x_aug, ((0, 0), (0, N_PAD - N_NODES), (0, 0)))

    (acc1,) = _make_sc_agg(HA, T1, G1, True, spmem_src=True)(
        x_aug, src_a, dst_a, jnp.zeros((B, HA), jnp.float32))

    grid = N_NODES // _R
    full = lambda i: (0, 0)
    z2, s2 = pl.pallas_call(
        _tc_mid_body,
        grid=(grid,),
        in_specs=[
            pl.BlockSpec((_R, D_IN), lambda i: (i, 0)),
            pl.BlockSpec((NC, _R, HA), lambda i: (0, i, 0)),
            pl.BlockSpec((D_IN, D_HID), full),
            pl.BlockSpec((D_IN, D_HID), full),
            pl.BlockSpec((1, D_HID), full),
            pl.BlockSpec((D_HID, D_OUT), full),
            pl.BlockSpec((D_HID, D_OUT), full),
            pl.BlockSpec((1, D_OUT), full),
        ],
        out_specs=[
            pl.BlockSpec((_R, D_OUT), lambda i: (i, 0)),
            pl.BlockSpec((_R, D_OUT), lambda i: (i, 0)),
        ],
        out_shape=[
            jax.ShapeDtypeStruct((N_NODES, D_OUT), jnp.float32),
            jax.ShapeDtypeStruct((N_NODES, D_OUT), jnp.float32),
        ],
    )(node_features, acc1, W_self1, W_neigh1, b1.reshape(1, D_HID),
      W_self2, W_neigh2, b2.reshape(1, D_OUT))

    z2_pad = jnp.pad(z2, ((0, N_PAD - N_NODES), (0, 0)))
    (acc2,) = _make_sc_agg(D_OUT, T2, G2, False, spmem_src=True)(
        z2_pad, src_b, dst_b, jnp.zeros((B, D_OUT), jnp.float32))

    out = pl.pallas_call(
        _tc_out_body,
        grid=(grid,),
        in_specs=[
            pl.BlockSpec((NC, _R, D_OUT), lambda i: (0, i, 0)),
            pl.BlockSpec((NC, _R, HA), lambda i: (0, i, 0)),
            pl.BlockSpec((_R, D_OUT), lambda i: (i, 0)),
        ],
        out_specs=pl.BlockSpec((_R, D_OUT), lambda i: (i, 0)),
        out_shape=jax.ShapeDtypeStruct((N_NODES, D_OUT), jnp.float32),
    )(acc2, acc1, s2)
    return out
